# baseline (speedup 1.0000x reference)
import jax
import jax.numpy as jnp
from jax import lax
from jax.experimental import pallas as pl
from jax.experimental.pallas import tpu as pltpu
from jax.experimental.pallas import tpu_sc as plsc

VOCAB = 1000000
D = 64
B = 16384
L = 50
N_IDX = B * L

NC = 2
NS = 16
NW = NC * NS

PAIR_ROWS = VOCAB // 2

VBLK = 128
NBLK = -(-VOCAB // VBLK)
KA_BASE = NBLK // NW
KA_EXTRA = NBLK - KA_BASE * NW
PAIR_ROWS_PAD = NBLK * (VBLK // 2)


def _ka_body(tT_hbm, trm_hbm, slab0, slab1, tb0, tb1, s0, s1, w0, w1):
    c = lax.axis_index("c")
    s = lax.axis_index("s")
    wid = s * NC + c
    nblk_w = KA_BASE + jnp.where(wid < KA_EXTRA, 1, 0)
    iota = lax.iota(jnp.int32, 16)

    def blk_of(k):
        return wid + NW * k

    def v0_of(blk):
        return pl.multiple_of(blk * VBLK, VBLK)

    def fire_slab(k, slab, sem):
        @pl.when(k < nblk_w)
        def _():
            pltpu.async_copy(
                tT_hbm.at[:, pl.ds(v0_of(blk_of(k)), VBLK)],
                slab.at[pl.ds(0, D), pl.ds(0, VBLK)], sem)

    def wait_slab(k, slab, sem):
        @pl.when(k < nblk_w)
        def _():
            pltpu.make_async_copy(
                tT_hbm.at[:, pl.ds(0, VBLK)],
                slab.at[pl.ds(0, D), pl.ds(0, VBLK)], sem).wait()

    def wait_wb(k_prev, tb, sem):
        @pl.when((k_prev >= 0) & (k_prev < nblk_w))
        def _():
            pltpu.make_async_copy(tb, trm_hbm.at[pl.ds(0, 64)], sem).wait()

    def select_store(k, slab, tb, sem):
        @pl.when(k < nblk_w)
        def _():
            def row(r, carry):
                for t in range(8):
                    rowv = iota + 16 * (t % 4)
                    colv = jnp.full((16,), 2 * r, jnp.int32) + (
                        1 if t >= 4 else 0)
                    tb[r, pl.ds(16 * t, 16)] = plsc.load_gather(
                        slab, [rowv, colv])
                return carry

            lax.fori_loop(0, 64, row, 0)
            u0 = pl.multiple_of(blk_of(k) * (VBLK // 2), VBLK // 2)
            pltpu.async_copy(tb, trm_hbm.at[pl.ds(u0, 64)], sem)

    def step(p, carry):
        k0 = 2 * p
        k1 = 2 * p + 1
        wait_wb(k0 - 2, tb0, w0)
        fire_slab(k0, slab0, s0)
        wait_wb(k1 - 2, tb1, w1)
        fire_slab(k1, slab1, s1)
        wait_slab(k0, slab0, s0)
        select_store(k0, slab0, tb0, w0)
        wait_slab(k1, slab1, s1)
        select_store(k1, slab1, tb1, w1)
        return carry

    lax.fori_loop(0, (KA_BASE + 2) // 2, step, 0)
    @pl.when(wid < KA_EXTRA)
    def _():
        wait_wb(0, tb0, w0)


B_BLK = 128
NBB = B // B_BLK
BB_PER_W = NBB // NW
XCHUNK = B_BLK * L


def _kb_body(x_hbm, trm_hbm, out_hbm, xv, idxb, offb, g0, g1, oc0, oc1,
             gs0, gs1, ws0, ws1):
    c = lax.axis_index("c")
    s = lax.axis_index("s")
    wid = s * NC + c
    iota = lax.iota(jnp.int32, 16)

    def fire_gather(l, gbuf, sem):
        pltpu.async_copy(
            trm_hbm.at[idxb.at[pl.ds(l * B_BLK, B_BLK)]],
            gbuf.at[:, pl.ds(0, 128)], sem)

    def wait_gather(gbuf, sem):
        pltpu.make_async_copy(
            trm_hbm.at[idxb.at[pl.ds(0, B_BLK)]],
            gbuf.at[:, pl.ds(0, 128)], sem).wait()

    def wait_wb(oc, sem):
        pltpu.make_async_copy(
            oc, out_hbm.at[0, :, pl.ds(0, B_BLK)], sem).wait()

    def select(l, b0, gbuf, oc, sem):
        for g in range(8):
            rowv = iota + 16 * g
            voff = offb[pl.ds(l * B_BLK + 16 * g, 16)]
            for d in range(D):
                colv = voff + d
                oc[d, pl.ds(16 * g, 16)] = plsc.load_gather(
                    gbuf, [rowv, colv])
        pltpu.async_copy(oc, out_hbm.at[l, :, pl.ds(b0, B_BLK)], sem)

    def block(m, carry):
        bblk = wid * BB_PER_W + m
        b0 = pl.multiple_of(bblk * B_BLK, B_BLK)
        pltpu.sync_copy(x_hbm.at[pl.ds(b0 * L, XCHUNK)], xv)

        def build(l, carry2):
            for g in range(8):
                addr = iota * L + (16 * g * L + l)
                v = plsc.load_gather(xv, [addr])
                idxb[pl.ds(l * B_BLK + 16 * g, 16)] = (
                    lax.shift_right_logical(v, 1))
                offb[pl.ds(l * B_BLK + 16 * g, 16)] = (
                    lax.shift_left(lax.bitwise_and(v, 1), 6))
            return carry2

        lax.fori_loop(0, L, build, 0)

        fire_gather(0, g0, gs0)
        fire_gather(1, g1, gs1)

        def pair(p, carry2):
            l0 = 2 * p
            l1 = l0 + 1
            wait_gather(g0, gs0)

            @pl.when(p > 0)
            def _():
                wait_wb(oc0, ws0)
            select(l0, b0, g0, oc0, ws0)

            @pl.when(l0 + 2 < L)
            def _():
                fire_gather(l0 + 2, g0, gs0)

            wait_gather(g1, gs1)

            @pl.when(p > 0)
            def _():
                wait_wb(oc1, ws1)
            select(l1, b0, g1, oc1, ws1)

            @pl.when(l1 + 2 < L)
            def _():
                fire_gather(l1 + 2, g1, gs1)
            return carry2

        lax.fori_loop(0, L // 2, pair, 0)
        wait_wb(oc0, ws0)
        wait_wb(oc1, ws1)
        return carry

    lax.fori_loop(0, BB_PER_W, block, 0)


def kernel(x, table):
    mesh = plsc.VectorSubcoreMesh(core_axis_name="c", subcore_axis_name="s")
    params = pltpu.CompilerParams(use_tc_tiling_on_sc=True,
                                  needs_layout_passes=False)

    trm = pl.kernel(
        _ka_body,
        out_type=jax.ShapeDtypeStruct((PAIR_ROWS_PAD, 128), jnp.float32),
        mesh=mesh,
        scratch_types=[
            pltpu.VMEM((VBLK, VBLK + 1), jnp.float32),
            pltpu.VMEM((VBLK, VBLK + 1), jnp.float32),
            pltpu.VMEM((64, 128), jnp.float32),
            pltpu.VMEM((64, 128), jnp.float32),
            pltpu.SemaphoreType.DMA,
            pltpu.SemaphoreType.DMA,
            pltpu.SemaphoreType.DMA,
            pltpu.SemaphoreType.DMA,
        ],
        compiler_params=params,
    )(table.T)

    out_t = pl.kernel(
        _kb_body,
        out_type=jax.ShapeDtypeStruct((L, D, B), jnp.float32),
        mesh=mesh,
        scratch_types=[
            pltpu.VMEM((XCHUNK,), jnp.int32),
            pltpu.VMEM((XCHUNK,), jnp.int32),
            pltpu.VMEM((XCHUNK,), jnp.int32),
            pltpu.VMEM((B_BLK, 129), jnp.float32),
            pltpu.VMEM((B_BLK, 129), jnp.float32),
            pltpu.VMEM((D, B_BLK), jnp.float32),
            pltpu.VMEM((D, B_BLK), jnp.float32),
            pltpu.SemaphoreType.DMA,
            pltpu.SemaphoreType.DMA,
            pltpu.SemaphoreType.DMA,
            pltpu.SemaphoreType.DMA,
        ],
        compiler_params=params,
    )(x.reshape(N_IDX).astype(jnp.int32), trm)

    return out_t.transpose(2, 0, 1)


# speedup vs baseline: 1.3690x; 1.3690x over previous
import jax
import jax.numpy as jnp
from jax import lax
from jax.experimental import pallas as pl
from jax.experimental.pallas import tpu as pltpu
from jax.experimental.pallas import tpu_sc as plsc

VOCAB = 1000000
D = 64
B = 16384
L = 50
N_IDX = B * L

NC = 2
NS = 16
NW = NC * NS

PAIR_ROWS = VOCAB // 2

VBLK = 128
NBLK = -(-VOCAB // VBLK)
KA_BASE = NBLK // NW
KA_EXTRA = NBLK - KA_BASE * NW
PAIR_ROWS_PAD = NBLK * (VBLK // 2)


def _ka_body(tT_hbm, trm_hbm, slab0, slab1, tb0, tb1, s0, s1, w0, w1):
    c = lax.axis_index("c")
    s = lax.axis_index("s")
    wid = s * NC + c
    nblk_w = KA_BASE + jnp.where(wid < KA_EXTRA, 1, 0)
    iota = lax.iota(jnp.int32, 16)

    def blk_of(k):
        return wid + NW * k

    def v0_of(blk):
        return pl.multiple_of(blk * VBLK, VBLK)

    def fire_slab(k, slab, sem):
        @pl.when(k < nblk_w)
        def _():
            pltpu.async_copy(
                tT_hbm.at[:, pl.ds(v0_of(blk_of(k)), VBLK)],
                slab.at[pl.ds(0, D), pl.ds(0, VBLK)], sem)

    def wait_slab(k, slab, sem):
        @pl.when(k < nblk_w)
        def _():
            pltpu.make_async_copy(
                tT_hbm.at[:, pl.ds(0, VBLK)],
                slab.at[pl.ds(0, D), pl.ds(0, VBLK)], sem).wait()

    def wait_wb(k_prev, tb, sem):
        @pl.when((k_prev >= 0) & (k_prev < nblk_w))
        def _():
            pltpu.make_async_copy(tb, trm_hbm.at[pl.ds(0, 64)], sem).wait()

    def select_store(k, slab, tb, sem):
        @pl.when(k < nblk_w)
        def _():
            def row(r, carry):
                c0 = jnp.full((16,), 2 * r, jnp.int32)
                c1 = c0 + 1
                vals = [
                    plsc.load_gather(slab,
                                     [iota + 16 * (t % 4),
                                      c1 if t >= 4 else c0])
                    for t in range(8)
                ]
                for t in range(8):
                    tb[r, pl.ds(16 * t, 16)] = vals[t]
                return carry

            lax.fori_loop(0, 64, row, 0)
            u0 = pl.multiple_of(blk_of(k) * (VBLK // 2), VBLK // 2)
            pltpu.async_copy(tb, trm_hbm.at[pl.ds(u0, 64)], sem)

    def step(p, carry):
        k0 = 2 * p
        k1 = 2 * p + 1
        wait_wb(k0 - 2, tb0, w0)
        fire_slab(k0, slab0, s0)
        wait_wb(k1 - 2, tb1, w1)
        fire_slab(k1, slab1, s1)
        wait_slab(k0, slab0, s0)
        select_store(k0, slab0, tb0, w0)
        wait_slab(k1, slab1, s1)
        select_store(k1, slab1, tb1, w1)
        return carry

    lax.fori_loop(0, (KA_BASE + 2) // 2, step, 0)
    @pl.when(wid < KA_EXTRA)
    def _():
        wait_wb(0, tb0, w0)


B_BLK = 128
NBB = B // B_BLK
BB_PER_W = NBB // NW
XCHUNK = B_BLK * L


def _kb_body(x_hbm, trm_hbm, out_hbm, xv, idxb, offb, g0, g1, oc0, oc1,
             gs0, gs1, ws0, ws1):
    c = lax.axis_index("c")
    s = lax.axis_index("s")
    wid = s * NC + c
    iota = lax.iota(jnp.int32, 16)

    def fire_gather(l, gbuf, sem):
        pltpu.async_copy(
            trm_hbm.at[idxb.at[pl.ds(l * B_BLK, B_BLK)]],
            gbuf.at[:, pl.ds(0, 128)], sem)

    def wait_gather(gbuf, sem):
        pltpu.make_async_copy(
            trm_hbm.at[idxb.at[pl.ds(0, B_BLK)]],
            gbuf.at[:, pl.ds(0, 128)], sem).wait()

    def wait_wb(oc, sem):
        pltpu.make_async_copy(
            oc, out_hbm.at[0, :, pl.ds(0, B_BLK)], sem).wait()

    def select(l, b0, gbuf, oc, sem):
        voffs = [offb[pl.ds(l * B_BLK + 16 * g, 16)] for g in range(8)]
        rows = [iota + 16 * g for g in range(8)]
        for d in range(D):
            vals = [
                plsc.load_gather(gbuf, [rows[g], voffs[g] + d])
                for g in range(8)
            ]
            for g in range(8):
                oc[d, pl.ds(16 * g, 16)] = vals[g]
        pltpu.async_copy(oc, out_hbm.at[l, :, pl.ds(b0, B_BLK)], sem)

    def block(m, carry):
        bblk = wid * BB_PER_W + m
        b0 = pl.multiple_of(bblk * B_BLK, B_BLK)
        pltpu.sync_copy(x_hbm.at[pl.ds(b0 * L, XCHUNK)], xv)

        def build(l, carry2):
            for g in range(8):
                addr = iota * L + (16 * g * L + l)
                v = plsc.load_gather(xv, [addr])
                idxb[pl.ds(l * B_BLK + 16 * g, 16)] = (
                    lax.shift_right_logical(v, 1))
                offb[pl.ds(l * B_BLK + 16 * g, 16)] = (
                    lax.shift_left(lax.bitwise_and(v, 1), 6))
            return carry2

        lax.fori_loop(0, L, build, 0)

        fire_gather(0, g0, gs0)
        fire_gather(1, g1, gs1)

        def pair(p, carry2):
            l0 = 2 * p
            l1 = l0 + 1
            wait_gather(g0, gs0)

            @pl.when(p > 0)
            def _():
                wait_wb(oc0, ws0)
            select(l0, b0, g0, oc0, ws0)

            @pl.when(l0 + 2 < L)
            def _():
                fire_gather(l0 + 2, g0, gs0)

            wait_gather(g1, gs1)

            @pl.when(p > 0)
            def _():
                wait_wb(oc1, ws1)
            select(l1, b0, g1, oc1, ws1)

            @pl.when(l1 + 2 < L)
            def _():
                fire_gather(l1 + 2, g1, gs1)
            return carry2

        lax.fori_loop(0, L // 2, pair, 0)
        wait_wb(oc0, ws0)
        wait_wb(oc1, ws1)
        return carry

    lax.fori_loop(0, BB_PER_W, block, 0)


def kernel(x, table):
    mesh = plsc.VectorSubcoreMesh(core_axis_name="c", subcore_axis_name="s")
    params = pltpu.CompilerParams(use_tc_tiling_on_sc=True,
                                  needs_layout_passes=False)

    trm = pl.kernel(
        _ka_body,
        out_type=jax.ShapeDtypeStruct((PAIR_ROWS_PAD, 128), jnp.float32),
        mesh=mesh,
        scratch_types=[
            pltpu.VMEM((VBLK, VBLK + 1), jnp.float32),
            pltpu.VMEM((VBLK, VBLK + 1), jnp.float32),
            pltpu.VMEM((64, 128), jnp.float32),
            pltpu.VMEM((64, 128), jnp.float32),
            pltpu.SemaphoreType.DMA,
            pltpu.SemaphoreType.DMA,
            pltpu.SemaphoreType.DMA,
            pltpu.SemaphoreType.DMA,
        ],
        compiler_params=params,
    )(table.T)

    out_t = pl.kernel(
        _kb_body,
        out_type=jax.ShapeDtypeStruct((L, D, B), jnp.float32),
        mesh=mesh,
        scratch_types=[
            pltpu.VMEM((XCHUNK,), jnp.int32),
            pltpu.VMEM((XCHUNK,), jnp.int32),
            pltpu.VMEM((XCHUNK,), jnp.int32),
            pltpu.VMEM((B_BLK, 129), jnp.float32),
            pltpu.VMEM((B_BLK, 129), jnp.float32),
            pltpu.VMEM((D, B_BLK), jnp.float32),
            pltpu.VMEM((D, B_BLK), jnp.float32),
            pltpu.SemaphoreType.DMA,
            pltpu.SemaphoreType.DMA,
            pltpu.SemaphoreType.DMA,
            pltpu.SemaphoreType.DMA,
        ],
        compiler_params=params,
    )(x.reshape(N_IDX).astype(jnp.int32), trm)

    return out_t.transpose(2, 0, 1)


# speedup vs baseline: 2.4063x; 1.7577x over previous
import jax
import jax.numpy as jnp
from jax import lax
from jax.experimental import pallas as pl
from jax.experimental.pallas import tpu as pltpu
from jax.experimental.pallas import tpu_sc as plsc

VOCAB = 1000000
D = 64
B = 16384
L = 50
N_IDX = B * L

NC = 2
NS = 16
NW = NC * NS
B_PER_W = B // NW

B_CHUNK = 16
CHUNK = B_CHUNK * L
N_CHUNKS = B_PER_W // B_CHUNK
N_PAIRS = N_CHUNKS // 2

GATHER_SIZES = [128] * (CHUNK // 128) + ([CHUNK % 128] if CHUNK % 128 else [])


def _fire_gathers(table_hbm, xv, rows_v, sem):
    cps = []
    off = 0
    for n in GATHER_SIZES:
        cps.append(pltpu.async_copy(
            table_hbm.at[xv.at[pl.ds(off, n)]],
            rows_v.at[pl.ds(off, n)],
            sem))
        off += n
    return cps


def _body(x_hbm, table_hbm, out_hbm, xv0, xv1, rows_v0, rows_v1,
          gsem0, gsem1, wsem0, wsem1):
    c = lax.axis_index("c")
    s = lax.axis_index("s")
    wid = s * NC + c
    b0w = wid * B_PER_W

    def _fire_wb(rows_v, b_base, sem):
        for bb in range(B_CHUNK):
            pltpu.async_copy(rows_v.at[pl.ds(bb * L, L)],
                             out_hbm.at[b_base + bb], sem)

    def _wait_wb(rows_v, sem):
        for bb in range(B_CHUNK):
            pltpu.make_async_copy(rows_v.at[pl.ds(bb * L, L)],
                                  out_hbm.at[b0w + bb], sem).wait()

    def step(p, carry):
        b0 = b0w + 2 * p * B_CHUNK
        b1 = b0 + B_CHUNK
        n0 = b0 * L
        n1 = n0 + CHUNK

        @pl.when(p > 0)
        def _():
            _wait_wb(rows_v0, wsem0)
        pltpu.sync_copy(x_hbm.at[pl.ds(n0, CHUNK)], xv0)
        g0 = _fire_gathers(table_hbm, xv0, rows_v0, gsem0)

        @pl.when(p > 0)
        def _():
            _wait_wb(rows_v1, wsem1)
        pltpu.sync_copy(x_hbm.at[pl.ds(n1, CHUNK)], xv1)
        g1 = _fire_gathers(table_hbm, xv1, rows_v1, gsem1)

        for cp in g0:
            cp.wait()
        _fire_wb(rows_v0, b0, wsem0)

        for cp in g1:
            cp.wait()
        _fire_wb(rows_v1, b1, wsem1)
        return carry

    lax.fori_loop(0, N_PAIRS, step, 0)
    _wait_wb(rows_v0, wsem0)
    _wait_wb(rows_v1, wsem1)


def kernel(x, table):
    x_flat = x.reshape(N_IDX).astype(jnp.int32)
    mesh = plsc.VectorSubcoreMesh(core_axis_name="c", subcore_axis_name="s")
    out = pl.kernel(
        _body,
        out_type=jax.ShapeDtypeStruct((B, L, D), jnp.float32),
        mesh=mesh,
        scratch_types=[
            pltpu.VMEM((CHUNK,), jnp.int32),
            pltpu.VMEM((CHUNK,), jnp.int32),
            pltpu.VMEM((CHUNK, D), jnp.float32),
            pltpu.VMEM((CHUNK, D), jnp.float32),
            pltpu.SemaphoreType.DMA,
            pltpu.SemaphoreType.DMA,
            pltpu.SemaphoreType.DMA,
            pltpu.SemaphoreType.DMA,
        ],
        compiler_params=pltpu.CompilerParams(use_tc_tiling_on_sc=False),
    )(x_flat, table)
    return out
